# Initial kernel scaffold; baseline (speedup 1.0000x reference)
import jax
import jax.numpy as jnp
from jax.experimental import pallas as pl


def kernel(x, weight):
    raise NotImplementedError("write your pallas kernel here")


# speedup vs baseline: 4.1574x; 4.1574x over previous
import functools

import jax
import jax.numpy as jnp
from jax import lax
from jax.experimental import pallas as pl
from jax.experimental.pallas import tpu as pltpu
from jax.experimental.pallas import tpu_sc as plsc

N = 8
VOCAB = 100000
DIM = 32
B = 1024
T = 20

NUM_IDX = B * T
NW = 32
Q = 4
PER_W = NUM_IDX // Q
CHUNK = 128
NCHUNK = PER_W // CHUNK
IDX_ROWS = NUM_IDX // CHUNK


def _body(x_hbm, w_hbm, out_hbm, idx_v, rows_v, sem):
    cid = lax.axis_index("c")
    sid = lax.axis_index("s")
    wid = sid * 2 + cid
    n = wid // Q
    q = wid % Q

    pltpu.sync_copy(x_hbm.at[pl.ds(q * NCHUNK, NCHUNK)], idx_v)

    off = jnp.full((16,), n * VOCAB, dtype=jnp.int32)

    def add_body(j, _):
        for i in range(CHUNK // 16):
            sl = (j, pl.ds(i * 16, 16))
            idx_v[sl] = idx_v[sl] + off
        return 0

    lax.fori_loop(0, NCHUNK, add_body, 0)

    out_base = wid * PER_W

    def gather_body(j, _):
        pltpu.async_copy(w_hbm.at[idx_v.at[j]], rows_v, sem).wait()
        pltpu.sync_copy(rows_v, out_hbm.at[pl.ds(out_base + j * CHUNK, CHUNK)])
        return 0

    lax.fori_loop(0, NCHUNK, gather_body, 0)


@jax.jit
def kernel(x, weight):
    x2d = jnp.reshape(x.astype(jnp.int32), (IDX_ROWS, CHUNK))
    w2d = jnp.reshape(weight, (N * VOCAB, DIM))
    call = pl.kernel(
        _body,
        mesh=plsc.VectorSubcoreMesh(core_axis_name="c", subcore_axis_name="s"),
        out_type=jax.ShapeDtypeStruct((N * NUM_IDX, DIM), jnp.float32),
        scratch_types=[
            pltpu.VMEM((NCHUNK, CHUNK), jnp.int32),
            pltpu.VMEM((CHUNK, DIM), jnp.float32),
            pltpu.SemaphoreType.DMA,
        ],
        compiler_params=pltpu.CompilerParams(use_tc_tiling_on_sc=False),
    )
    out = call(x2d, w2d)
    return jnp.reshape(out, (N, B, T, DIM))


# speedup vs baseline: 4.3942x; 1.0570x over previous
import jax
import jax.numpy as jnp
from jax import lax
from jax.experimental import pallas as pl
from jax.experimental.pallas import tpu as pltpu
from jax.experimental.pallas import tpu_sc as plsc

N = 8
VOCAB = 100000
DIM = 32
B = 1024
T = 20

NUM_IDX = B * T
NW = 32
Q = 4
PER_W = NUM_IDX // Q
CHUNK = 128
NCHUNK = PER_W // CHUNK
IDX_ROWS = NUM_IDX // CHUNK
R = 8
NGRP = NCHUNK // R


def _body(x_hbm, w_hbm, out_hbm, idx_v, rows_v, gsem, wsem):
    cid = lax.axis_index("c")
    sid = lax.axis_index("s")
    wid = sid * 2 + cid
    n = wid // Q
    q = wid % Q

    pltpu.sync_copy(x_hbm.at[pl.ds(q * NCHUNK, NCHUNK)], idx_v)

    off = jnp.full((16,), n * VOCAB, dtype=jnp.int32)

    def add_chunk(j):
        for i in range(CHUNK // 16):
            sl = (j, pl.ds(i * 16, 16))
            idx_v[sl] = idx_v[sl] + off

    out_base = wid * PER_W

    def fire_gather(j, b):
        pltpu.async_copy(w_hbm.at[idx_v.at[j]], rows_v.at[b], gsem.at[b])

    def wait_gather(b):
        pltpu.make_async_copy(w_hbm.at[idx_v.at[0]], rows_v.at[b],
                              gsem.at[b]).wait()

    def fire_write(j, b):
        pltpu.async_copy(rows_v.at[b],
                         out_hbm.at[pl.ds(out_base + j * CHUNK, CHUNK)],
                         wsem.at[b])

    def wait_write(b):
        pltpu.make_async_copy(w_hbm.at[idx_v.at[0]], rows_v.at[b],
                              wsem.at[b]).wait()

    for b in range(R):
        add_chunk(b)
        fire_gather(b, b)

    def group_body(g, _):
        for b in range(R):
            wait_gather(b)
            fire_write(g * R + b, b)
        for b in range(R):
            add_chunk((g + 1) * R + b)
        for b in range(R):
            wait_write(b)
            fire_gather((g + 1) * R + b, b)
        return 0

    lax.fori_loop(0, NGRP - 1, group_body, 0)

    for b in range(R):
        wait_gather(b)
        fire_write((NGRP - 1) * R + b, b)
    for b in range(R):
        wait_write(b)


@jax.jit
def kernel(x, weight):
    x2d = jnp.reshape(x.astype(jnp.int32), (IDX_ROWS, CHUNK))
    w2d = jnp.reshape(weight, (N * VOCAB, DIM))
    call = pl.kernel(
        _body,
        mesh=plsc.VectorSubcoreMesh(core_axis_name="c", subcore_axis_name="s"),
        out_type=jax.ShapeDtypeStruct((N * NUM_IDX, DIM), jnp.float32),
        scratch_types=[
            pltpu.VMEM((NCHUNK, CHUNK), jnp.int32),
            pltpu.VMEM((R, CHUNK, DIM), jnp.float32),
            pltpu.SemaphoreType.DMA((R,)),
            pltpu.SemaphoreType.DMA((R,)),
        ],
        compiler_params=pltpu.CompilerParams(use_tc_tiling_on_sc=False),
    )
    out = call(x2d, w2d)
    return jnp.reshape(out, (N, B, T, DIM))


# speedup vs baseline: 10.2586x; 2.3346x over previous
import jax
import jax.numpy as jnp
from jax import lax
from jax.experimental import pallas as pl
from jax.experimental.pallas import tpu as pltpu
from jax.experimental.pallas import tpu_sc as plsc

N = 8
VOCAB = 100000
DIM = 32
B = 1024
T = 20

NUM_IDX = B * T
NW = 32
CHUNK = 128
NCHUNK = NUM_IDX // CHUNK
PER_W = NCHUNK // NW
ND = N * DIM
R = 3


def _body(x_hbm, w_hbm, out_hbm, idx_v, rows_v, gsem, wsem):
    cid = lax.axis_index("c")
    sid = lax.axis_index("s")
    wid = sid * 2 + cid
    base = wid * PER_W

    start = jnp.minimum((base // 8) * 8, NCHUNK - 16)
    loc = base - start
    pltpu.sync_copy(x_hbm.at[pl.ds(start, 16)], idx_v)

    def fire_gather(j):
        pltpu.async_copy(w_hbm.at[idx_v.at[loc + j]], rows_v.at[j % R],
                         gsem.at[j % R])

    def wait_gather(j):
        pltpu.make_async_copy(w_hbm.at[idx_v.at[loc]], rows_v.at[j % R],
                              gsem.at[j % R]).wait()

    def fire_write(j):
        pltpu.async_copy(rows_v.at[j % R],
                         out_hbm.at[pl.ds((base + j) * CHUNK, CHUNK)],
                         wsem.at[j % R])

    def wait_write(j):
        pltpu.make_async_copy(w_hbm.at[idx_v.at[loc]], rows_v.at[j % R],
                              wsem.at[j % R]).wait()

    for j in range(R):
        fire_gather(j)
    for j in range(PER_W):
        wait_gather(j)
        fire_write(j)
        if j + R < PER_W:
            wait_write(j)
            fire_gather(j + R)
    for j in range(PER_W - R, PER_W):
        wait_write(j)


@jax.jit
def kernel(x, weight):
    x2d = jnp.reshape(x.astype(jnp.int32), (NCHUNK, CHUNK))
    w2d = jnp.reshape(jnp.transpose(weight, (1, 0, 2)), (VOCAB, ND))
    call = pl.kernel(
        _body,
        mesh=plsc.VectorSubcoreMesh(core_axis_name="c", subcore_axis_name="s"),
        out_type=jax.ShapeDtypeStruct((NUM_IDX, ND), jnp.float32),
        scratch_types=[
            pltpu.VMEM((16, CHUNK), jnp.int32),
            pltpu.VMEM((R, CHUNK, ND), jnp.float32),
            pltpu.SemaphoreType.DMA((R,)),
            pltpu.SemaphoreType.DMA((R,)),
        ],
        compiler_params=pltpu.CompilerParams(use_tc_tiling_on_sc=True),
    )
    out = call(x2d, w2d)
    return jnp.transpose(jnp.reshape(out, (B, T, N, DIM)), (2, 0, 1, 3))
